# baseline (speedup 1.0000x reference)
import functools

import jax
import jax.numpy as jnp
from jax import lax
from jax.experimental import pallas as pl
from jax.experimental.pallas import tpu as pltpu
from jax.experimental.pallas import tpu_sc as plsc

F32 = jnp.float32
H = 128
NC = 2
NS = 16
NW = NC * NS


def _silu(x):
    return x * (1.0 / (1.0 + jnp.exp(-x)))


def _tc_matmul(x, w):
    n = x.shape[0]
    bn = min(n, 2048)

    def body(x_ref, w_ref, o_ref):
        o_ref[...] = jnp.dot(x_ref[...], w_ref[...], preferred_element_type=F32)

    return pl.pallas_call(
        body,
        grid=(n // bn,),
        in_specs=[
            pl.BlockSpec((bn, H), lambda i: (i, 0)),
            pl.BlockSpec((H, H), lambda i: (0, 0)),
        ],
        out_specs=pl.BlockSpec((bn, H), lambda i: (i, 0)),
        out_shape=jax.ShapeDtypeStruct((n, H), F32),
    )(x, w)


def _tc_edge_mlp(edge, m_prev, gs, gd, w1e, b1, w2, b2):
    e = edge.shape[0]
    be = min(e, 2048)
    has_prev = m_prev is not None

    def body(*refs):
        if has_prev:
            e_ref, mp_ref, gs_ref, gd_ref, w1_ref, b1_ref, w2_ref, b2_ref, o_ref = refs
            x = e_ref[...] + mp_ref[...]
        else:
            e_ref, gs_ref, gd_ref, w1_ref, b1_ref, w2_ref, b2_ref, o_ref = refs
            x = e_ref[...]
        pre = (
            jnp.dot(x, w1_ref[...], preferred_element_type=F32)
            + gs_ref[...]
            + gd_ref[...]
            + b1_ref[...]
        )
        h = _silu(pre)
        o_ref[...] = jnp.dot(h, w2_ref[...], preferred_element_type=F32) + b2_ref[...]

    row_spec = pl.BlockSpec((be, H), lambda i: (i, 0))
    w_spec = pl.BlockSpec((H, H), lambda i: (0, 0))
    b_spec = pl.BlockSpec((1, H), lambda i: (0, 0))
    in_specs = [row_spec] + ([row_spec] if has_prev else []) + [
        row_spec, row_spec, w_spec, b_spec, w_spec, b_spec]
    args = [edge] + ([m_prev] if has_prev else []) + [
        gs, gd, w1e, b1.reshape(1, H), w2, b2.reshape(1, H)]
    return pl.pallas_call(
        body,
        grid=(e // be,),
        in_specs=in_specs,
        out_specs=row_spec,
        out_shape=jax.ShapeDtypeStruct((e, H), F32),
    )(*args)


def _tc_node_mlp(rec, a0, a1, extra, v1r, v1a, c1, v2, c2):
    n = rec.shape[0]
    bn = min(n, 2048)
    has_extra = extra is not None

    def body(*refs):
        if has_extra:
            r_ref, a0_ref, a1_ref, x_ref, v1r_ref, v1a_ref, c1_ref, v2_ref, c2_ref, o_ref = refs
        else:
            r_ref, a0_ref, a1_ref, v1r_ref, v1a_ref, c1_ref, v2_ref, c2_ref, o_ref = refs
        rec_v = r_ref[...]
        aggr = a0_ref[...] + a1_ref[...]
        pre = (
            jnp.dot(rec_v, v1r_ref[...], preferred_element_type=F32)
            + jnp.dot(aggr, v1a_ref[...], preferred_element_type=F32)
            + c1_ref[...]
        )
        out = rec_v + jnp.dot(_silu(pre), v2_ref[...], preferred_element_type=F32) + c2_ref[...]
        if has_extra:
            out = out + x_ref[...]
        o_ref[...] = out

    row_spec = pl.BlockSpec((bn, H), lambda i: (i, 0))
    w_spec = pl.BlockSpec((H, H), lambda i: (0, 0))
    b_spec = pl.BlockSpec((1, H), lambda i: (0, 0))
    in_specs = [row_spec, row_spec, row_spec] + ([row_spec] if has_extra else []) + [
        w_spec, w_spec, b_spec, w_spec, b_spec]
    args = [rec, a0, a1] + ([extra] if has_extra else []) + [
        v1r, v1a, c1.reshape(1, H), v2, c2.reshape(1, H)]
    return pl.pallas_call(
        body,
        grid=(n // bn,),
        in_specs=in_specs,
        out_specs=row_spec,
        out_shape=jax.ShapeDtypeStruct((n, H), F32),
    )(*args)


def _tc_final(rep, lpm):
    (w1, b1), (w2, b2) = lpm
    n = rep.shape[0]
    lat_dim = w2.shape[1]
    half = lat_dim // 2

    def body(r_ref, w1_ref, b1_ref, w2_ref, b2_ref, mu_ref, std_ref):
        h = _silu(jnp.dot(r_ref[...], w1_ref[...], preferred_element_type=F32) + b1_ref[...])
        lat = jnp.dot(h, w2_ref[...], preferred_element_type=F32) + b2_ref[...]
        mu_ref[...] = lat[:, :half]
        raw = lat[:, half:]
        sp = jnp.maximum(raw, 0.0) + jnp.log(1.0 + jnp.exp(-jnp.abs(raw)))
        std_ref[...] = 1e-4 + sp

    return pl.pallas_call(
        body,
        in_specs=[
            pl.BlockSpec((n, H), lambda: (0, 0)),
            pl.BlockSpec((H, H), lambda: (0, 0)),
            pl.BlockSpec((1, H), lambda: (0, 0)),
            pl.BlockSpec((H, lat_dim), lambda: (0, 0)),
            pl.BlockSpec((1, lat_dim), lambda: (0, 0)),
        ],
        out_specs=[
            pl.BlockSpec((n, half), lambda: (0, 0)),
            pl.BlockSpec((n, half), lambda: (0, 0)),
        ],
        out_shape=[
            jax.ShapeDtypeStruct((n, half), F32),
            jax.ShapeDtypeStruct((n, half), F32),
        ],
    )(rep, w1, b1.reshape(1, H), w2, b2.reshape(1, lat_dim))


def _sc_mesh():
    return plsc.VectorSubcoreMesh(
        core_axis_name="c", subcore_axis_name="s", num_cores=NC, num_subcores=NS)


_CH = 128


def _sc_gather_multi(jobs):
    nsubs = [j[2].shape[0] // NW for j in jobs]
    tot = sum(nsubs)
    out_type = []
    for j in jobs:
        e = j[2].shape[0] * _CH
        out_type += [jax.ShapeDtypeStruct((e, H), F32)] * 2
    scratch = [pltpu.VMEM((tot, _CH), jnp.int32), pltpu.VMEM((tot, _CH), jnp.int32),
               pltpu.VMEM((_CH, H), F32), pltpu.VMEM((_CH, H), F32),
               pltpu.SemaphoreType.DMA, pltpu.SemaphoreType.DMA]

    @functools.partial(pl.kernel, out_type=tuple(out_type), mesh=_sc_mesh(),
                       scratch_types=scratch)
    def k(*refs):
        nj = len(jobs)
        ins = refs[:4 * nj]
        outs = refs[4 * nj:4 * nj + 2 * nj]
        si, di, rs, rd, sem1, sem2 = refs[6 * nj:]
        cid = lax.axis_index("c")
        sid = lax.axis_index("s")
        wid = sid * NC + cid

        base = 0
        for ji in range(nj):
            sp_hbm, dp_hbm, s_hbm, d_hbm = ins[4 * ji:4 * ji + 4]
            gs_hbm, gd_hbm = outs[2 * ji:2 * ji + 2]
            nsub = nsubs[ji]
            row0 = wid * nsub
            pltpu.sync_copy(s_hbm.at[pl.ds(row0, nsub)], si.at[pl.ds(base, nsub)])
            pltpu.sync_copy(d_hbm.at[pl.ds(row0, nsub)], di.at[pl.ds(base, nsub)])

            def body(j, c, base=base, row0=row0, sp_hbm=sp_hbm, dp_hbm=dp_hbm,
                     gs_hbm=gs_hbm, gd_hbm=gd_hbm):
                c1 = pltpu.async_copy(sp_hbm.at[si.at[base + j]], rs, sem1)
                c2 = pltpu.async_copy(dp_hbm.at[di.at[base + j]], rd, sem2)
                c1.wait()
                c2.wait()
                pltpu.sync_copy(rs, gs_hbm.at[pl.ds((row0 + j) * _CH, _CH)])
                pltpu.sync_copy(rd, gd_hbm.at[pl.ds((row0 + j) * _CH, _CH)])
                return c

            lax.fori_loop(0, nsub, body, 0)
            base += nsub

    flat = []
    for j in jobs:
        flat += list(j)
    return k(*flat)


def _sc_scatter_multi(jobs):
    nsubs = [j[1].shape[0] // NW for j in jobs]
    tot = sum(nsubs)
    zrs = [j[2] // NS for j in jobs]
    zmax = max(zrs)
    out_type = tuple(jax.ShapeDtypeStruct((NC * j[2], H), F32) for j in jobs)
    scratch = [pltpu.VMEM((tot, _CH), jnp.int32),
               pltpu.VMEM((_CH, H), F32),
               pltpu.VMEM((zmax, H), F32)]
    scratch += [pltpu.VMEM_SHARED((j[2], H), F32) for j in jobs]

    @functools.partial(pl.kernel, out_type=out_type, mesh=_sc_mesh(),
                       scratch_types=scratch)
    def k(*refs):
        nj = len(jobs)
        ins = refs[:2 * nj]
        outs = refs[2 * nj:3 * nj]
        di = refs[3 * nj]
        rows = refs[3 * nj + 1]
        zbuf = refs[3 * nj + 2]
        tables = refs[3 * nj + 3:3 * nj + 3 + nj]
        cid = lax.axis_index("c")
        sid = lax.axis_index("s")

        def zero_body(i, c):
            zbuf[i // (H // 16), pl.ds((i % (H // 16)) * 16, 16)] = jnp.zeros((16,), F32)
            return c

        lax.fori_loop(0, zmax * (H // 16), zero_body, 0)
        for ji in range(nj):
            pltpu.sync_copy(zbuf.at[pl.ds(0, zrs[ji])],
                            tables[ji].at[pl.ds(sid * zrs[ji], zrs[ji])])
        plsc.subcore_barrier()

        base = 0
        for ji in range(nj):
            m_hbm, d_hbm = ins[2 * ji:2 * ji + 2]
            nsub = nsubs[ji]
            row0 = cid * (nsub * NS) + sid * nsub
            pltpu.sync_copy(d_hbm.at[pl.ds(row0, nsub)], di.at[pl.ds(base, nsub)])

            def body(j, c, nsub=nsub, base=base, row0=row0, m_hbm=m_hbm,
                     table=tables[ji]):
                pltpu.sync_copy(m_hbm.at[pl.ds((row0 + j) * _CH, _CH)], rows)
                pltpu.sync_copy(rows, table.at[di.at[base + j]], add=True)
                return c

            lax.fori_loop(0, nsub, body, 0)
            base += nsub
        plsc.subcore_barrier()

        for ji in range(nj):
            pltpu.sync_copy(tables[ji].at[pl.ds(sid * zrs[ji], zrs[ji])],
                            zbuf.at[pl.ds(0, zrs[ji])])
            pltpu.sync_copy(zbuf.at[pl.ds(0, zrs[ji])],
                            outs[ji].at[pl.ds(cid * jobs[ji][2] + sid * zrs[ji], zrs[ji])])

    flat = []
    for j in jobs:
        flat += [j[0], j[1]]
    return k(*flat)


def _wsplit(p):
    (w1, b1), (w2, b2) = p["edge"]
    (v1, c1), (v2, c2) = p["node"]
    return dict(w1e=w1[:H], w1s=w1[H:2 * H], w1d=w1[2 * H:], b1=b1, w2=w2,
                b2=b2, v1r=v1[:H], v1a=v1[H:], c1=c1, v2=v2, c2=c2)


def _gnn_layer(p, send, rec, edge, m_prev, src2d, dst2d, extra=None, need_m=False):
    w = _wsplit(p)
    sp = _tc_matmul(send, w["w1s"])
    dpp = _tc_matmul(rec, w["w1d"])
    gs, gd = _sc_gather_multi([(sp, dpp, src2d, dst2d)])
    m = _tc_edge_mlp(edge, m_prev, gs, gd, w["w1e"], w["b1"], w["w2"], w["b2"])
    n_rec = rec.shape[0]
    (ag,) = _sc_scatter_multi([(m, dst2d, n_rec)])
    rec_new = _tc_node_mlp(rec, ag[:n_rec], ag[n_rec:], extra,
                           w["v1r"], w["v1a"], w["c1"], w["v2"], w["c2"])
    return rec_new, (m if need_m else None)


def _gnn_pair(pa, pb, ja, jb):
    wa, wb = _wsplit(pa), _wsplit(pb)
    spa = _tc_matmul(ja["send"], wa["w1s"])
    dpa = _tc_matmul(ja["rec"], wa["w1d"])
    spb = _tc_matmul(jb["send"], wb["w1s"])
    dpb = _tc_matmul(jb["rec"], wb["w1d"])
    gsa, gda, gsb, gdb = _sc_gather_multi([
        (spa, dpa, ja["s"], ja["d"]), (spb, dpb, jb["s"], jb["d"])])
    ma = _tc_edge_mlp(ja["edge"], ja.get("m_prev"), gsa, gda,
                      wa["w1e"], wa["b1"], wa["w2"], wa["b2"])
    mb = _tc_edge_mlp(jb["edge"], jb.get("m_prev"), gsb, gdb,
                      wb["w1e"], wb["b1"], wb["w2"], wb["b2"])
    na, nb = ja["rec"].shape[0], jb["rec"].shape[0]
    aga, agb = _sc_scatter_multi([(ma, ja["d"], na), (mb, jb["d"], nb)])
    ra = _tc_node_mlp(ja["rec"], aga[:na], aga[na:], ja.get("extra"),
                      wa["v1r"], wa["v1a"], wa["c1"], wa["v2"], wa["c2"])
    xb = ra if jb.get("extra") == "A" else jb.get("extra")
    rb = _tc_node_mlp(jb["rec"], agb[:nb], agb[nb:], xb,
                      wb["v1r"], wb["v1a"], wb["c1"], wb["v2"], wb["c2"])
    return (ra, ma if ja.get("need_m") else None,
            rb, mb if jb.get("need_m") else None)


def _ei2d(ei):
    e = ei.shape[1]
    s = ei[0].astype(jnp.int32).reshape(e // _CH, _CH)
    d = ei[1].astype(jnp.int32).reshape(e // _CH, _CH)
    return s, d


def kernel(high_emb, low_emb, hr_mesh_0, hr_mesh_1, hr_g2m_feat, hr_m2m_feat_0,
           hr_m2m_feat_1, hr_mesh_up_feat_0, lr_mesh_0, lr_g2m_feat,
           lr_m2m_feat_0, params, hr_g2m_edge_index, hr_m2m_edge_index_0,
           hr_m2m_edge_index_1, hr_mesh_up_edge_index_0, lr_g2m_edge_index,
           lr_m2m_edge_index_0):
    he = high_emb[0]
    le = low_emb[0]
    hm0 = hr_mesh_0[0]
    hm1 = hr_mesh_1[0]
    g2m_feat = hr_g2m_feat[0]
    m2m0 = hr_m2m_feat_0[0]
    m2m1 = hr_m2m_feat_1[0]
    upf = hr_mesh_up_feat_0[0]
    lm0 = lr_mesh_0[0]
    lrg2m = lr_g2m_feat[0]
    lrm2m = lr_m2m_feat_0[0]

    s_g2m, d_g2m = _ei2d(hr_g2m_edge_index)
    s_m2m0, d_m2m0 = _ei2d(hr_m2m_edge_index_0)
    s_m2m1, d_m2m1 = _ei2d(hr_m2m_edge_index_1)
    s_up, d_up = _ei2d(hr_mesh_up_edge_index_0)
    s_lg2m, d_lg2m = _ei2d(lr_g2m_edge_index)
    s_lm2m, d_lm2m = _ei2d(lr_m2m_edge_index_0)

    P = params

    hr_rep, _, lr_up, _ = _gnn_pair(
        P["g2m"], P["cond_g2m"],
        dict(send=he, rec=hm0, edge=g2m_feat, s=s_g2m, d=d_g2m),
        dict(send=le, rec=lm0, edge=lrg2m, s=s_lg2m, d=d_lg2m))
    n1, m1, n2, m2 = _gnn_pair(
        P["intra0"][0], P["cond_intra0"][0],
        dict(send=hr_rep, rec=hr_rep, edge=m2m0, s=s_m2m0, d=d_m2m0,
             need_m=True),
        dict(send=lr_up, rec=lr_up, edge=lrm2m, s=s_lm2m, d=d_lm2m,
             need_m=True))
    rep, _, lr_in, _ = _gnn_pair(
        P["intra0"][1], P["cond_intra0"][1],
        dict(send=n1, rec=n1, edge=m2m0, m_prev=m1, s=s_m2m0, d=d_m2m0,
             extra=le),
        dict(send=n2, rec=n2, edge=lrm2m, m_prev=m2, s=s_lm2m, d=d_lm2m))

    rep2, _ = _gnn_layer(P["up0"], rep, hm1, upf, None, s_up, d_up, extra=lr_up)

    n3, m3 = _gnn_layer(P["intra1"][0], rep2, rep2, m2m1, None,
                        s_m2m1, d_m2m1, need_m=True)
    rep3, _ = _gnn_layer(P["intra1"][1], n3, n3, m2m1, m3, s_m2m1, d_m2m1,
                         extra=lr_in)

    mu, std = _tc_final(rep3, P["lpm"])
    return (mu[None], std[None], low_emb, lr_in[None], lr_up[None])


# speedup vs baseline: 1.0739x; 1.0739x over previous
import functools

import jax
import jax.numpy as jnp
from jax import lax
from jax.experimental import pallas as pl
from jax.experimental.pallas import tpu as pltpu
from jax.experimental.pallas import tpu_sc as plsc

F32 = jnp.float32
H = 128
NC = 2
NS = 16
NW = NC * NS


def _silu(x):
    return x * (1.0 / (1.0 + jnp.exp(-x)))


def _tc_matmul(x, w):
    n = x.shape[0]
    bn = min(n, 2048)

    def body(x_ref, w_ref, o_ref):
        o_ref[...] = jnp.dot(x_ref[...], w_ref[...], preferred_element_type=F32)

    return pl.pallas_call(
        body,
        grid=(n // bn,),
        in_specs=[
            pl.BlockSpec((bn, H), lambda i: (i, 0)),
            pl.BlockSpec((H, H), lambda i: (0, 0)),
        ],
        out_specs=pl.BlockSpec((bn, H), lambda i: (i, 0)),
        out_shape=jax.ShapeDtypeStruct((n, H), F32),
    )(x, w)


def _tc_edge_mlp(edge, m_prev, gs, gd, w1e, b1, w2, b2):
    e = edge.shape[0]
    be = min(e, 2048)
    has_prev = m_prev is not None

    def body(*refs):
        if has_prev:
            e_ref, mp_ref, gs_ref, gd_ref, w1_ref, b1_ref, w2_ref, b2_ref, o_ref = refs
            x = e_ref[...] + mp_ref[...]
        else:
            e_ref, gs_ref, gd_ref, w1_ref, b1_ref, w2_ref, b2_ref, o_ref = refs
            x = e_ref[...]
        pre = (
            jnp.dot(x, w1_ref[...], preferred_element_type=F32)
            + gs_ref[...]
            + gd_ref[...]
            + b1_ref[...]
        )
        h = _silu(pre)
        o_ref[...] = jnp.dot(h, w2_ref[...], preferred_element_type=F32) + b2_ref[...]

    row_spec = pl.BlockSpec((be, H), lambda i: (i, 0))
    w_spec = pl.BlockSpec((H, H), lambda i: (0, 0))
    b_spec = pl.BlockSpec((1, H), lambda i: (0, 0))
    in_specs = [row_spec] + ([row_spec] if has_prev else []) + [
        row_spec, row_spec, w_spec, b_spec, w_spec, b_spec]
    args = [edge] + ([m_prev] if has_prev else []) + [
        gs, gd, w1e, b1.reshape(1, H), w2, b2.reshape(1, H)]
    return pl.pallas_call(
        body,
        grid=(e // be,),
        in_specs=in_specs,
        out_specs=row_spec,
        out_shape=jax.ShapeDtypeStruct((e, H), F32),
    )(*args)


def _tc_node_mlp(rec, a0, a1, extra, v1r, v1a, c1, v2, c2):
    n = rec.shape[0]
    bn = min(n, 2048)
    has_extra = extra is not None

    def body(*refs):
        if has_extra:
            r_ref, a0_ref, a1_ref, x_ref, v1r_ref, v1a_ref, c1_ref, v2_ref, c2_ref, o_ref = refs
        else:
            r_ref, a0_ref, a1_ref, v1r_ref, v1a_ref, c1_ref, v2_ref, c2_ref, o_ref = refs
        rec_v = r_ref[...]
        aggr = a0_ref[...] + a1_ref[...]
        pre = (
            jnp.dot(rec_v, v1r_ref[...], preferred_element_type=F32)
            + jnp.dot(aggr, v1a_ref[...], preferred_element_type=F32)
            + c1_ref[...]
        )
        out = rec_v + jnp.dot(_silu(pre), v2_ref[...], preferred_element_type=F32) + c2_ref[...]
        if has_extra:
            out = out + x_ref[...]
        o_ref[...] = out

    row_spec = pl.BlockSpec((bn, H), lambda i: (i, 0))
    w_spec = pl.BlockSpec((H, H), lambda i: (0, 0))
    b_spec = pl.BlockSpec((1, H), lambda i: (0, 0))
    in_specs = [row_spec, row_spec, row_spec] + ([row_spec] if has_extra else []) + [
        w_spec, w_spec, b_spec, w_spec, b_spec]
    args = [rec, a0, a1] + ([extra] if has_extra else []) + [
        v1r, v1a, c1.reshape(1, H), v2, c2.reshape(1, H)]
    return pl.pallas_call(
        body,
        grid=(n // bn,),
        in_specs=in_specs,
        out_specs=row_spec,
        out_shape=jax.ShapeDtypeStruct((n, H), F32),
    )(*args)


def _tc_final(rep, lpm):
    (w1, b1), (w2, b2) = lpm
    n = rep.shape[0]
    lat_dim = w2.shape[1]
    half = lat_dim // 2

    def body(r_ref, w1_ref, b1_ref, w2_ref, b2_ref, mu_ref, std_ref):
        h = _silu(jnp.dot(r_ref[...], w1_ref[...], preferred_element_type=F32) + b1_ref[...])
        lat = jnp.dot(h, w2_ref[...], preferred_element_type=F32) + b2_ref[...]
        mu_ref[...] = lat[:, :half]
        raw = lat[:, half:]
        sp = jnp.maximum(raw, 0.0) + jnp.log(1.0 + jnp.exp(-jnp.abs(raw)))
        std_ref[...] = 1e-4 + sp

    return pl.pallas_call(
        body,
        in_specs=[
            pl.BlockSpec((n, H), lambda: (0, 0)),
            pl.BlockSpec((H, H), lambda: (0, 0)),
            pl.BlockSpec((1, H), lambda: (0, 0)),
            pl.BlockSpec((H, lat_dim), lambda: (0, 0)),
            pl.BlockSpec((1, lat_dim), lambda: (0, 0)),
        ],
        out_specs=[
            pl.BlockSpec((n, half), lambda: (0, 0)),
            pl.BlockSpec((n, half), lambda: (0, 0)),
        ],
        out_shape=[
            jax.ShapeDtypeStruct((n, half), F32),
            jax.ShapeDtypeStruct((n, half), F32),
        ],
    )(rep, w1, b1.reshape(1, H), w2, b2.reshape(1, lat_dim))


def _sc_mesh():
    return plsc.VectorSubcoreMesh(
        core_axis_name="c", subcore_axis_name="s", num_cores=NC, num_subcores=NS)


_CH = 128


def _sc_gather_multi(jobs):
    nsubs = [j[2].shape[0] // NW for j in jobs]
    tot = sum(nsubs)
    out_type = []
    for j in jobs:
        e = j[2].shape[0] * _CH
        out_type += [jax.ShapeDtypeStruct((e, H), F32)] * 2
    scratch = [pltpu.VMEM((tot, _CH), jnp.int32), pltpu.VMEM((tot, _CH), jnp.int32),
               pltpu.VMEM((_CH, H), F32), pltpu.VMEM((_CH, H), F32),
               pltpu.SemaphoreType.DMA, pltpu.SemaphoreType.DMA]

    @functools.partial(pl.kernel, out_type=tuple(out_type), mesh=_sc_mesh(),
                       scratch_types=scratch)
    def k(*refs):
        nj = len(jobs)
        ins = refs[:4 * nj]
        outs = refs[4 * nj:4 * nj + 2 * nj]
        si, di, rs, rd, sem1, sem2 = refs[6 * nj:]
        cid = lax.axis_index("c")
        sid = lax.axis_index("s")
        wid = sid * NC + cid

        base = 0
        for ji in range(nj):
            sp_hbm, dp_hbm, s_hbm, d_hbm = ins[4 * ji:4 * ji + 4]
            gs_hbm, gd_hbm = outs[2 * ji:2 * ji + 2]
            nsub = nsubs[ji]
            row0 = wid * nsub
            pltpu.sync_copy(s_hbm.at[pl.ds(row0, nsub)], si.at[pl.ds(base, nsub)])
            pltpu.sync_copy(d_hbm.at[pl.ds(row0, nsub)], di.at[pl.ds(base, nsub)])

            def body(j, c, base=base, row0=row0, sp_hbm=sp_hbm, dp_hbm=dp_hbm,
                     gs_hbm=gs_hbm, gd_hbm=gd_hbm):
                c1 = pltpu.async_copy(sp_hbm.at[si.at[base + j]], rs, sem1)
                c2 = pltpu.async_copy(dp_hbm.at[di.at[base + j]], rd, sem2)
                c1.wait()
                c2.wait()
                pltpu.sync_copy(rs, gs_hbm.at[pl.ds((row0 + j) * _CH, _CH)])
                pltpu.sync_copy(rd, gd_hbm.at[pl.ds((row0 + j) * _CH, _CH)])
                return c

            lax.fori_loop(0, nsub, body, 0)
            base += nsub

    flat = []
    for j in jobs:
        flat += list(j)
    return k(*flat)


def _sc_scatter_multi(jobs):
    nsubs = [j[1].shape[0] // NW for j in jobs]
    tot = sum(nsubs)
    zrs = [j[2] // NS for j in jobs]
    zmax = max(zrs)
    out_type = tuple(jax.ShapeDtypeStruct((NC * j[2], H), F32) for j in jobs)
    scratch = [pltpu.VMEM((tot, _CH), jnp.int32),
               pltpu.VMEM((_CH, H), F32),
               pltpu.VMEM((zmax, H), F32)]
    scratch += [pltpu.VMEM_SHARED((j[2], H), F32) for j in jobs]

    @functools.partial(pl.kernel, out_type=out_type, mesh=_sc_mesh(),
                       scratch_types=scratch)
    def k(*refs):
        nj = len(jobs)
        ins = refs[:2 * nj]
        outs = refs[2 * nj:3 * nj]
        di = refs[3 * nj]
        rows = refs[3 * nj + 1]
        zbuf = refs[3 * nj + 2]
        tables = refs[3 * nj + 3:3 * nj + 3 + nj]
        cid = lax.axis_index("c")
        sid = lax.axis_index("s")

        def zero_body(i, c):
            zbuf[i // (H // 16), pl.ds((i % (H // 16)) * 16, 16)] = jnp.zeros((16,), F32)
            return c

        lax.fori_loop(0, zmax * (H // 16), zero_body, 0)
        for ji in range(nj):
            pltpu.sync_copy(zbuf.at[pl.ds(0, zrs[ji])],
                            tables[ji].at[pl.ds(sid * zrs[ji], zrs[ji])])
        plsc.subcore_barrier()

        base = 0
        for ji in range(nj):
            m_hbm, d_hbm = ins[2 * ji:2 * ji + 2]
            nsub = nsubs[ji]
            row0 = cid * (nsub * NS) + sid * nsub
            pltpu.sync_copy(d_hbm.at[pl.ds(row0, nsub)], di.at[pl.ds(base, nsub)])

            def body(j, c, nsub=nsub, base=base, row0=row0, m_hbm=m_hbm,
                     table=tables[ji]):
                pltpu.sync_copy(m_hbm.at[pl.ds((row0 + j) * _CH, _CH)], rows)
                pltpu.sync_copy(rows, table.at[di.at[base + j]], add=True)
                return c

            lax.fori_loop(0, nsub, body, 0)
            base += nsub
        plsc.subcore_barrier()

        for ji in range(nj):
            pltpu.sync_copy(tables[ji].at[pl.ds(sid * zrs[ji], zrs[ji])],
                            zbuf.at[pl.ds(0, zrs[ji])])
            pltpu.sync_copy(zbuf.at[pl.ds(0, zrs[ji])],
                            outs[ji].at[pl.ds(cid * jobs[ji][2] + sid * zrs[ji], zrs[ji])])

    flat = []
    for j in jobs:
        flat += [j[0], j[1]]
    return k(*flat)


def _wsplit(p):
    (w1, b1), (w2, b2) = p["edge"]
    (v1, c1), (v2, c2) = p["node"]
    return dict(w1e=w1[:H], w1s=w1[H:2 * H], w1d=w1[2 * H:], b1=b1, w2=w2,
                b2=b2, v1r=v1[:H], v1a=v1[H:], c1=c1, v2=v2, c2=c2)


def _gnn_layer(p, send, rec, edge, m_prev, src2d, dst2d, extra=None, need_m=False):
    w = _wsplit(p)
    sp = _tc_matmul(send, w["w1s"])
    dpp = _tc_matmul(rec, w["w1d"])
    gs, gd = _sc_gather_multi([(sp, dpp, src2d, dst2d)])
    m = _tc_edge_mlp(edge, m_prev, gs, gd, w["w1e"], w["b1"], w["w2"], w["b2"])
    n_rec = rec.shape[0]
    (ag,) = _sc_scatter_multi([(m, dst2d, n_rec)])
    rec_new = _tc_node_mlp(rec, ag[:n_rec], ag[n_rec:], extra,
                           w["v1r"], w["v1a"], w["c1"], w["v2"], w["c2"])
    return rec_new, (m if need_m else None)


def _gnn_pair(pa, pb, ja, jb):
    wa, wb = _wsplit(pa), _wsplit(pb)
    spa = _tc_matmul(ja["send"], wa["w1s"])
    dpa = _tc_matmul(ja["rec"], wa["w1d"])
    spb = _tc_matmul(jb["send"], wb["w1s"])
    dpb = _tc_matmul(jb["rec"], wb["w1d"])
    gsa, gda, gsb, gdb = _sc_gather_multi([
        (spa, dpa, ja["s"], ja["d"]), (spb, dpb, jb["s"], jb["d"])])
    ma = _tc_edge_mlp(ja["edge"], ja.get("m_prev"), gsa, gda,
                      wa["w1e"], wa["b1"], wa["w2"], wa["b2"])
    mb = _tc_edge_mlp(jb["edge"], jb.get("m_prev"), gsb, gdb,
                      wb["w1e"], wb["b1"], wb["w2"], wb["b2"])
    na, nb = ja["rec"].shape[0], jb["rec"].shape[0]
    aga, agb = _sc_scatter_multi([(ma, ja["d"], na), (mb, jb["d"], nb)])
    ra = _tc_node_mlp(ja["rec"], aga[:na], aga[na:], ja.get("extra"),
                      wa["v1r"], wa["v1a"], wa["c1"], wa["v2"], wa["c2"])
    xb = ra if jb.get("extra") == "A" else jb.get("extra")
    rb = _tc_node_mlp(jb["rec"], agb[:nb], agb[nb:], xb,
                      wb["v1r"], wb["v1a"], wb["c1"], wb["v2"], wb["c2"])
    return (ra, ma if ja.get("need_m") else None,
            rb, mb if jb.get("need_m") else None)


def _ei2d(ei):
    e = ei.shape[1]
    s = ei[0].astype(jnp.int32).reshape(e // _CH, _CH)
    d = ei[1].astype(jnp.int32).reshape(e // _CH, _CH)
    return s, d


def kernel(high_emb, low_emb, hr_mesh_0, hr_mesh_1, hr_g2m_feat, hr_m2m_feat_0,
           hr_m2m_feat_1, hr_mesh_up_feat_0, lr_mesh_0, lr_g2m_feat,
           lr_m2m_feat_0, params, hr_g2m_edge_index, hr_m2m_edge_index_0,
           hr_m2m_edge_index_1, hr_mesh_up_edge_index_0, lr_g2m_edge_index,
           lr_m2m_edge_index_0):
    he = high_emb[0]
    le = low_emb[0]
    hm0 = hr_mesh_0[0]
    hm1 = hr_mesh_1[0]
    g2m_feat = hr_g2m_feat[0]
    m2m0 = hr_m2m_feat_0[0]
    m2m1 = hr_m2m_feat_1[0]
    upf = hr_mesh_up_feat_0[0]
    lm0 = lr_mesh_0[0]
    lrg2m = lr_g2m_feat[0]
    lrm2m = lr_m2m_feat_0[0]

    s_g2m, d_g2m = _ei2d(hr_g2m_edge_index)
    s_m2m0, d_m2m0 = _ei2d(hr_m2m_edge_index_0)
    s_m2m1, d_m2m1 = _ei2d(hr_m2m_edge_index_1)
    s_up, d_up = _ei2d(hr_mesh_up_edge_index_0)
    s_lg2m, d_lg2m = _ei2d(lr_g2m_edge_index)
    s_lm2m, d_lm2m = _ei2d(lr_m2m_edge_index_0)

    P = params

    hr_rep, _ = _gnn_layer(P["g2m"], he, hm0, g2m_feat, None, s_g2m, d_g2m)
    n1, m1 = _gnn_layer(P["intra0"][0], hr_rep, hr_rep, m2m0, None,
                        s_m2m0, d_m2m0, need_m=True)
    rep, _ = _gnn_layer(P["intra0"][1], n1, n1, m2m0, m1, s_m2m0, d_m2m0,
                        extra=le)

    lr_up, _ = _gnn_layer(P["cond_g2m"], le, lm0, lrg2m, None, s_lg2m, d_lg2m)
    rep2, _ = _gnn_layer(P["up0"], rep, hm1, upf, None, s_up, d_up, extra=lr_up)

    n2, m2 = _gnn_layer(P["cond_intra0"][0], lr_up, lr_up, lrm2m, None,
                        s_lm2m, d_lm2m, need_m=True)
    lr_in, _ = _gnn_layer(P["cond_intra0"][1], n2, n2, lrm2m, m2,
                          s_lm2m, d_lm2m)

    n3, m3 = _gnn_layer(P["intra1"][0], rep2, rep2, m2m1, None,
                        s_m2m1, d_m2m1, need_m=True)
    rep3, _ = _gnn_layer(P["intra1"][1], n3, n3, m2m1, m3, s_m2m1, d_m2m1,
                         extra=lr_in)

    mu, std = _tc_final(rep3, P["lpm"])
    return (mu[None], std[None], low_emb, lr_in[None], lr_up[None])
